# baseline (speedup 1.0000x reference)
import functools

import jax
import jax.numpy as jnp
from jax.experimental import pallas as pl
from jax.experimental.pallas import tpu as pltpu


def _erf(x):
    p = jnp.float32(0.3275911)
    a1 = jnp.float32(0.254829592)
    a2 = jnp.float32(-0.284496736)
    a3 = jnp.float32(1.421413741)
    a4 = jnp.float32(-1.453152027)
    a5 = jnp.float32(1.061405429)
    ax = jnp.abs(x)
    t = 1.0 / (1.0 + p * ax)
    poly = t * (a1 + t * (a2 + t * (a3 + t * (a4 + t * a5))))
    y = 1.0 - poly * jnp.exp(-ax * ax)
    return jnp.sign(x) * y


def _gelu(x):
    return x * 0.5 * (1.0 + _erf(x * jnp.float32(0.7071067811865476)))


def _mlp_kernel(x_ref, ids_ref, w1_ref, b1_ref, w2_ref, b2_ref, w3_ref, b3_ref,
                ds_ref, do_ref, pb_ref, so_ref):
    S = x_ref.shape[0]
    E = w1_ref.shape[0]
    x = x_ref[...]
    ids = ids_ref[...]
    eiota = jax.lax.broadcasted_iota(jnp.int32, (S, E), 1)
    onehot = (ids == eiota).astype(jnp.float32)

    def dot_t(a, w):
        return jax.lax.dot_general(a, w, (((1,), (1,)), ((), ())),
                                   preferred_element_type=jnp.float32)

    pb = jnp.zeros(pb_ref.shape, jnp.float32)
    for e in range(E):
        h1 = _gelu(dot_t(x, w1_ref[e]) + b1_ref[e])
        h2 = _gelu(dot_t(h1, w2_ref[e]) + b2_ref[e])
        eo = dot_t(h2, w3_ref[e]) + b3_ref[e]
        pb = pb + onehot[:, e:e + 1] * eo
    pb_ref[...] = pb

    counts = jnp.sum(onehot, axis=0, keepdims=True)
    inv_s = jnp.float32(1.0 / S)
    avg_scale = jnp.dot(counts, ds_ref[...], preferred_element_type=jnp.float32) * inv_s
    avg_offset = jnp.dot(counts, do_ref[...], preferred_element_type=jnp.float32) * inv_s
    so_ref[0:1, :] = avg_scale
    so_ref[1:2, :] = avg_offset


def _bias_kernel(so_ref, pb_ref, o_ref, *, tile_i):
    h = pl.program_id(0)
    ib = pl.program_id(1)
    S = o_ref.shape[2]
    c = so_ref[0, h] * jnp.float32(0.01)
    off = so_ref[1, h]
    base = ib * tile_i
    rows = jax.lax.broadcasted_iota(jnp.int32, (tile_i, S), 0) + base
    cols = jax.lax.broadcasted_iota(jnp.int32, (tile_i, S), 1)
    rel = jnp.abs(rows - cols).astype(jnp.float32)
    o_ref[0] = pb_ref[0] * jnp.exp(off - c * rel)


def kernel(scalars, seq_len, inscription_ids, W1, b1, W2, b2, W3, b3,
           distance_scales, distance_offsets):
    del seq_len
    B, S, D = scalars.shape
    E, HID, _ = W1.shape
    H = W3.shape[1]

    x = scalars.reshape(S, D)
    ids = inscription_ids.reshape(S, 1).astype(jnp.int32)

    pb, so = pl.pallas_call(
        _mlp_kernel,
        out_shape=(
            jax.ShapeDtypeStruct((S, H), jnp.float32),
            jax.ShapeDtypeStruct((2, H), jnp.float32),
        ),
    )(x, ids, W1, b1, W2, b2, W3, b3, distance_scales, distance_offsets)

    pb3 = pb.T.reshape(H, S, 1)

    TILE_I = 256
    grid = (H, S // TILE_I)
    bias = pl.pallas_call(
        functools.partial(_bias_kernel, tile_i=TILE_I),
        grid=grid,
        in_specs=[
            pl.BlockSpec(memory_space=pltpu.SMEM),
            pl.BlockSpec((1, TILE_I, 1), lambda h, i: (h, i, 0)),
        ],
        out_specs=pl.BlockSpec((1, TILE_I, S), lambda h, i: (h, i, 0)),
        out_shape=jax.ShapeDtypeStruct((H, S, S), jnp.float32),
    )(so, pb3)

    return bias.reshape(B, H, S, S)
